# baseline (speedup 1.0000x reference)
import jax
import jax.numpy as jnp
from jax import lax
from jax.experimental import pallas as pl
from jax.experimental.pallas import tpu as pltpu
from jax.experimental.pallas import tpu_sc as plsc

DIM = 8192
NCLS = 1024
NROW = 8192
NQ = 2048

NCORE = 2
NSUB = 16
LANES = 16

UNIT = 128
KU = 4
CW = KU * UNIT
NPASS = DIM // (NCORE * CW)
ROWS_PER_TILE = NROW // NSUB
RB = 32
NBLK = ROWS_PER_TILE // RB
CLS_PER_TILE = NCLS // NSUB


def _sc_body(inp_hbm, idx_hbm, cnt_hbm, n_hbm,
             a00, a01, a02, a03, a10, a11, a12, a13, nacc,
             ra0, ra1, ra2, ra3, rb0, rb1, rb2, rb3,
             zbuf, idxa, idxb, onesbuf,
             sema, semb, semw0, semw1, semz0, semz1):
    c = lax.axis_index("c")
    s = lax.axis_index("s")
    accb = [[a00, a01, a02, a03], [a10, a11, a12, a13]]
    banks = [[ra0, ra1, ra2, ra3], [rb0, rb1, rb2, rb3]]
    idxs = [idxa, idxb]
    sems = [sema, semb]
    semw = [semw0, semw1]
    semz = [semz0, semz1]

    ones_v = jnp.full((LANES,), 1.0, jnp.float32)
    zero_v = jnp.zeros((LANES,), jnp.float32)
    for i in range(RB):
        for j in range(UNIT // LANES):
            onesbuf[i, pl.ds(j * LANES, LANES)] = ones_v
    for i in range(CLS_PER_TILE):
        for j in range(UNIT // LANES):
            zbuf[i, pl.ds(j * LANES, LANES)] = zero_v

    cls0 = s * CLS_PER_TILE

    def prefetch(p, b, bank):
        r0 = s * ROWS_PER_TILE + b * RB
        ds = [pltpu.async_copy(idx_hbm.at[pl.ds(r0, RB)], idxs[bank],
                               sems[bank])]
        for u in range(KU):
            col0 = ((p * NCORE + c) * KU + u) * UNIT
            ds.append(pltpu.async_copy(
                inp_hbm.at[pl.ds(r0, RB), pl.ds(col0, UNIT)],
                banks[bank][u], sems[bank]))
        return ds

    def fire_zero(ab):
        return [pltpu.async_copy(zbuf, accb[ab][u].at[pl.ds(cls0, CLS_PER_TILE)],
                                 semz[ab]) for u in range(KU)]

    nz = pltpu.async_copy(zbuf, nacc.at[pl.ds(cls0, CLS_PER_TILE)], semz0)
    zd = [fire_zero(0), fire_zero(1)]
    nz.wait()
    wd = [None, None]

    for p in range(NPASS):
        ab = p % 2
        pend = prefetch(p, 0, 0)
        for dsc in zd[ab]:
            dsc.wait()
        zd[ab] = None
        plsc.subcore_barrier()

        for b in range(NBLK):
            bank = b % 2
            nxt = None
            if b + 1 < NBLK:
                nxt = prefetch(p, b + 1, 1 - bank)
            for dsc in pend:
                dsc.wait()
            pend = nxt
            for u in range(KU):
                pltpu.sync_copy(banks[bank][u], accb[ab][u].at[idxs[bank]],
                                add=True)
            if p == 0:
                pltpu.sync_copy(onesbuf, nacc.at[idxs[bank]], add=True)

        plsc.subcore_barrier()

        wd[ab] = []
        for u in range(KU):
            col0 = ((p * NCORE + c) * KU + u) * UNIT
            wd[ab].append(pltpu.async_copy(
                accb[ab][u].at[pl.ds(cls0, CLS_PER_TILE)],
                cnt_hbm.at[pl.ds(cls0, CLS_PER_TILE), pl.ds(col0, UNIT)],
                semw[ab]))
        if p == 0:
            pltpu.sync_copy(nacc.at[pl.ds(cls0, CLS_PER_TILE)],
                            n_hbm.at[pl.ds(cls0, CLS_PER_TILE)])
        ob = 1 - ab
        if wd[ob] is not None:
            for dsc in wd[ob]:
                dsc.wait()
            wd[ob] = None
            if p + 1 < NPASS:
                zd[ob] = fire_zero(ob)

    for dsc in wd[(NPASS - 1) % 2]:
        dsc.wait()


def _make_sc_train():
    mesh = plsc.VectorSubcoreMesh(core_axis_name="c", subcore_axis_name="s")
    rbt = pltpu.VMEM((RB, UNIT), jnp.float32)
    acct = pltpu.VMEM_SHARED((NCLS, UNIT), jnp.float32)
    return pl.kernel(
        _sc_body,
        out_type=(
            jax.ShapeDtypeStruct((NCLS, DIM), jnp.float32),
            jax.ShapeDtypeStruct((NCLS, UNIT), jnp.float32),
        ),
        mesh=mesh,
        scratch_types=[
            acct, acct, acct, acct,
            acct, acct, acct, acct,
            acct,
            rbt, rbt, rbt, rbt,
            rbt, rbt, rbt, rbt,
            pltpu.VMEM((CLS_PER_TILE, UNIT), jnp.float32),
            pltpu.VMEM((RB,), jnp.int32),
            pltpu.VMEM((RB,), jnp.int32),
            pltpu.VMEM((RB, UNIT), jnp.float32),
            pltpu.SemaphoreType.DMA,
            pltpu.SemaphoreType.DMA,
            pltpu.SemaphoreType.DMA,
            pltpu.SemaphoreType.DMA,
            pltpu.SemaphoreType.DMA,
            pltpu.SemaphoreType.DMA,
        ],
    )


CB = 128


def _thr_body(c_ref, n_ref, a_ref):
    cnt = c_ref[...]
    n = n_ref[...][:, :1]
    a_ref[...] = jnp.where(cnt + cnt > n, 1.0, 0.0).astype(jnp.bfloat16)


def _tc_threshold(counts, ncnt):
    return pl.pallas_call(
        _thr_body,
        grid=(NCLS // CB,),
        in_specs=[
            pl.BlockSpec((CB, DIM), lambda i: (i, 0)),
            pl.BlockSpec((CB, UNIT), lambda i: (i, 0)),
        ],
        out_specs=pl.BlockSpec((CB, DIM), lambda i: (i, 0)),
        out_shape=jax.ShapeDtypeStruct((NCLS, DIM), jnp.bfloat16),
    )(counts, ncnt)


BM = 256


def _tc_body(q_ref, a_ref, o_ref):
    qf = q_ref[...]
    q = qf.astype(jnp.bfloat16)
    a = a_ref[...]
    acc = lax.dot_general(q, a, (((1,), (1,)), ((), ())),
                          preferred_element_type=jnp.float32)
    qs = jnp.sum(qf, axis=1)
    asum = jnp.sum(a.astype(jnp.float32), axis=1)
    o_ref[...] = 2.0 * acc - qs[:, None] - asum[None, :] + float(DIM)


def _tc_search(q, ab):
    return pl.pallas_call(
        _tc_body,
        grid=(NQ // BM,),
        in_specs=[
            pl.BlockSpec((BM, DIM), lambda i: (i, 0)),
            pl.BlockSpec((NCLS, DIM), lambda i: (0, 0)),
        ],
        out_specs=pl.BlockSpec((BM, NCLS), lambda i: (i, 0)),
        out_shape=jax.ShapeDtypeStruct((NQ, NCLS), jnp.float32),
    )(q, ab)


@jax.jit
def kernel(input, idx, query):
    counts, ncnt = _make_sc_train()(input, idx)
    am = _tc_threshold(counts, ncnt)
    return _tc_search(query, am)


# speedup vs baseline: 1.0235x; 1.0235x over previous
import jax
import jax.numpy as jnp
from jax import lax
from jax.experimental import pallas as pl
from jax.experimental.pallas import tpu as pltpu
from jax.experimental.pallas import tpu_sc as plsc

DIM = 8192
NCLS = 1024
NROW = 8192
NQ = 2048

NCORE = 2
NSUB = 16
LANES = 16

UNIT = 128
KU = 4
CW = KU * UNIT
NPASS = DIM // (NCORE * CW)
ROWS_PER_TILE = NROW // NSUB
RB = 64
NBLK = ROWS_PER_TILE // RB
CLS_PER_TILE = NCLS // NSUB


def _sc_body(inp_hbm, idx_hbm, cnt_hbm, n_hbm,
             acc0, acc1, acc2, acc3, nacc,
             ra0, ra1, ra2, ra3, rb0, rb1, rb2, rb3,
             zbuf, idxa, idxb, onesbuf, sema, semb, semz, sems0, sems1):
    c = lax.axis_index("c")
    s = lax.axis_index("s")
    accs = [acc0, acc1, acc2, acc3]
    banks = [[ra0, ra1, ra2, ra3], [rb0, rb1, rb2, rb3]]
    idxs = [idxa, idxb]
    sems = [sema, semb]
    sem_s = [sems0, sems1]

    ones_v = jnp.full((LANES,), 1.0, jnp.float32)
    zero_v = jnp.zeros((LANES,), jnp.float32)
    for i in range(RB):
        for j in range(UNIT // LANES):
            onesbuf[i, pl.ds(j * LANES, LANES)] = ones_v
            zbuf[i, pl.ds(j * LANES, LANES)] = zero_v

    cls0 = s * CLS_PER_TILE

    d = pltpu.async_copy(zbuf, nacc.at[pl.ds(cls0, CLS_PER_TILE)], semz)
    d.wait()

    @pl.loop(0, NPASS)
    def _pass(p):
        zd = [pltpu.async_copy(zbuf, accs[u].at[pl.ds(cls0, CLS_PER_TILE)],
                               semz) for u in range(KU)]

        def prefetch(b, bank):
            r0 = s * ROWS_PER_TILE + b * RB
            ds = [pltpu.async_copy(idx_hbm.at[pl.ds(r0, RB)], idxs[bank],
                                   sems[bank])]
            for u in range(KU):
                col0 = ((p * NCORE + c) * KU + u) * UNIT
                ds.append(pltpu.async_copy(
                    inp_hbm.at[pl.ds(r0, RB), pl.ds(col0, UNIT)],
                    banks[bank][u], sems[bank]))
            return ds

        pend = prefetch(0, 0)
        for u in range(KU):
            zd[u].wait()
        plsc.subcore_barrier()

        scat = [None, None]
        for b in range(NBLK):
            bank = b % 2
            nxt = None
            if b + 1 < NBLK:
                if scat[1 - bank] is not None:
                    for dsc in scat[1 - bank]:
                        dsc.wait()
                    scat[1 - bank] = None
                nxt = prefetch(b + 1, 1 - bank)
            for dsc in pend:
                dsc.wait()
            pend = nxt
            sd = [pltpu.async_copy(banks[bank][u], accs[u].at[idxs[bank]],
                                   sem_s[bank], add=True)
                  for u in range(KU)]

            @pl.when(p == 0)
            def _():
                pltpu.sync_copy(onesbuf, nacc.at[idxs[bank]], add=True)

            scat[bank] = sd
        for bank in range(2):
            if scat[bank] is not None:
                for dsc in scat[bank]:
                    dsc.wait()

        plsc.subcore_barrier()

        wd = []
        for u in range(KU):
            col0 = ((p * NCORE + c) * KU + u) * UNIT
            wd.append(pltpu.async_copy(
                accs[u].at[pl.ds(cls0, CLS_PER_TILE)],
                cnt_hbm.at[pl.ds(cls0, CLS_PER_TILE), pl.ds(col0, UNIT)],
                semz))
        for dsc in wd:
            dsc.wait()

        @pl.when(p == 0)
        def _():
            pltpu.sync_copy(nacc.at[pl.ds(cls0, CLS_PER_TILE)],
                            n_hbm.at[pl.ds(cls0, CLS_PER_TILE)])


def _make_sc_train():
    mesh = plsc.VectorSubcoreMesh(core_axis_name="c", subcore_axis_name="s")
    rbt = pltpu.VMEM((RB, UNIT), jnp.float32)
    return pl.kernel(
        _sc_body,
        out_type=(
            jax.ShapeDtypeStruct((NCLS, DIM), jnp.float32),
            jax.ShapeDtypeStruct((NCLS, UNIT), jnp.float32),
        ),
        mesh=mesh,
        scratch_types=[
            pltpu.VMEM_SHARED((NCLS, UNIT), jnp.float32),
            pltpu.VMEM_SHARED((NCLS, UNIT), jnp.float32),
            pltpu.VMEM_SHARED((NCLS, UNIT), jnp.float32),
            pltpu.VMEM_SHARED((NCLS, UNIT), jnp.float32),
            pltpu.VMEM_SHARED((NCLS, UNIT), jnp.float32),
            rbt, rbt, rbt, rbt,
            rbt, rbt, rbt, rbt,
            pltpu.VMEM((RB, UNIT), jnp.float32),
            pltpu.VMEM((RB,), jnp.int32),
            pltpu.VMEM((RB,), jnp.int32),
            pltpu.VMEM((RB, UNIT), jnp.float32),
            pltpu.SemaphoreType.DMA,
            pltpu.SemaphoreType.DMA,
            pltpu.SemaphoreType.DMA,
            pltpu.SemaphoreType.DMA,
            pltpu.SemaphoreType.DMA,
        ],
    )


CB = 128


def _thr_body(c_ref, n_ref, a_ref):
    cnt = c_ref[...]
    n = n_ref[...][:, :1]
    a_ref[...] = jnp.where(cnt + cnt > n, 1.0, 0.0).astype(jnp.bfloat16)


def _tc_threshold(counts, ncnt):
    return pl.pallas_call(
        _thr_body,
        grid=(NCLS // CB,),
        in_specs=[
            pl.BlockSpec((CB, DIM), lambda i: (i, 0)),
            pl.BlockSpec((CB, UNIT), lambda i: (i, 0)),
        ],
        out_specs=pl.BlockSpec((CB, DIM), lambda i: (i, 0)),
        out_shape=jax.ShapeDtypeStruct((NCLS, DIM), jnp.bfloat16),
    )(counts, ncnt)


BM = 256


def _tc_body(q_ref, a_ref, o_ref):
    qf = q_ref[...]
    q = qf.astype(jnp.bfloat16)
    a = a_ref[...]
    acc = lax.dot_general(q, a, (((1,), (1,)), ((), ())),
                          preferred_element_type=jnp.float32)
    qs = jnp.sum(qf, axis=1)
    asum = jnp.sum(a.astype(jnp.float32), axis=1)
    o_ref[...] = 2.0 * acc - qs[:, None] - asum[None, :] + float(DIM)


def _tc_search(q, ab):
    return pl.pallas_call(
        _tc_body,
        grid=(NQ // BM,),
        in_specs=[
            pl.BlockSpec((BM, DIM), lambda i: (i, 0)),
            pl.BlockSpec((NCLS, DIM), lambda i: (0, 0)),
        ],
        out_specs=pl.BlockSpec((BM, NCLS), lambda i: (i, 0)),
        out_shape=jax.ShapeDtypeStruct((NQ, NCLS), jnp.float32),
    )(q, ab)


@jax.jit
def kernel(input, idx, query):
    counts, ncnt = _make_sc_train()(input, idx)
    am = _tc_threshold(counts, ncnt)
    return _tc_search(query, am)
